# baseline (speedup 1.0000x reference)
import functools

import jax
import jax.numpy as jnp
from jax import lax
from jax.experimental import pallas as pl
from jax.experimental.pallas import tpu as pltpu
from jax.experimental.pallas import tpu_sc as plsc

T = 2048
HD = 768
NE = 8
FF = 4 * HD
BLK = 128
PAD = T * 2 + NE * BLK
NBLK = PAD // BLK
NW = 32
TPW = T // NW
LANES = 16


def _router_body(x_ref, gw_ref, gb_ref, pos_ref, w_ref, be_ref, laux_ref,
                 counts_ref):
    xf = x_ref[...]
    logits = jnp.dot(xf, gw_ref[...],
                     preferred_element_type=jnp.float32) + gb_ref[...]

    idx8 = lax.broadcasted_iota(jnp.int32, (T, NE), 1)
    v0 = jnp.max(logits, axis=1, keepdims=True)
    i0 = jnp.min(jnp.where(logits == v0, idx8, NE), axis=1, keepdims=True)
    masked = jnp.where(idx8 == i0, -jnp.inf, logits)
    v1 = jnp.max(masked, axis=1, keepdims=True)
    i1 = jnp.min(jnp.where(masked == v1, idx8, NE), axis=1, keepdims=True)

    e1 = jnp.exp(v1 - v0)
    denom = 1.0 + e1
    w_ref[:, 0:1] = 1.0 / denom
    w_ref[:, 1:2] = e1 / denom

    p = jnp.exp(logits - v0)
    probs = p / jnp.sum(p, axis=1, keepdims=True)
    pm = jnp.sum(probs, axis=0, keepdims=True) * (1.0 / T)
    laux_ref[...] = jnp.sum(pm * pm, axis=1, keepdims=True) * NE

    oh0 = (idx8 == i0).astype(jnp.float32)
    oh1 = (idx8 == i1).astype(jnp.float32)
    oht = oh0 + oh1

    cum = oht
    sh = 1
    while sh < T:
        cum = cum + jnp.concatenate(
            [jnp.zeros((sh, NE), jnp.float32), cum[:T - sh]], axis=0)
        sh *= 2
    counts = cum[T - 1:T, :]
    counts_ref[...] = counts
    cum_excl = jnp.concatenate(
        [jnp.zeros((1, NE), jnp.float32), cum[:T - 1]], axis=0)

    ci = counts.astype(jnp.int32)
    pc = ((ci + (BLK - 1)) // BLK) * BLK
    oc = pc
    sh = 1
    while sh < NE:
        oc = oc + jnp.concatenate(
            [jnp.zeros((1, sh), jnp.int32), oc[:, :NE - sh]], axis=1)
        sh *= 2
    offs = jnp.concatenate(
        [jnp.zeros((1, 1), jnp.int32), oc[:, :NE - 1]], axis=1)

    row0 = lax.broadcasted_iota(jnp.int32, (NBLK, NE), 0) * BLK
    offs_b = jnp.broadcast_to(offs, (NBLK, NE))
    pc_b = jnp.broadcast_to(pc, (NBLK, NE))
    e_iota = lax.broadcasted_iota(jnp.int32, (NBLK, NE), 1)
    ind = (row0 >= offs_b) & (row0 < offs_b + pc_b)
    be_ref[...] = jnp.sum(jnp.where(ind, e_iota, 0), axis=1, keepdims=True)

    posmat = jnp.broadcast_to(offs.astype(jnp.float32), (T, NE)) + cum_excl
    pos0 = jnp.sum(jnp.where(idx8 == i0, posmat, 0.0), axis=1, keepdims=True)
    pos1 = jnp.sum(jnp.where(idx8 == i1, posmat, 0.0), axis=1, keepdims=True)
    pos_ref[:, 0:1] = pos0.astype(jnp.int32)
    pos_ref[:, 1:2] = pos1.astype(jnp.int32)


def _run_router(xf, gate_w, gate_b):
    return pl.pallas_call(
        _router_body,
        out_shape=[
            jax.ShapeDtypeStruct((T, 2), jnp.int32),
            jax.ShapeDtypeStruct((T, 2), jnp.float32),
            jax.ShapeDtypeStruct((NBLK, 1), jnp.int32),
            jax.ShapeDtypeStruct((1, 1), jnp.float32),
            jax.ShapeDtypeStruct((1, NE), jnp.float32),
        ],
    )(xf, gate_w, gate_b.reshape(1, NE))


_SC_MESH = plsc.VectorSubcoreMesh(core_axis_name="c", subcore_axis_name="s",
                                  num_cores=2, num_subcores=16)


@functools.partial(
    pl.kernel,
    out_type=jax.ShapeDtypeStruct((PAD, HD // 2), jnp.int32),
    mesh=_SC_MESH,
    scratch_types=[
        pltpu.VMEM((TPW, HD // 2), jnp.int32),
        pltpu.VMEM((TPW,), jnp.int32),
        pltpu.VMEM((TPW,), jnp.int32),
        pltpu.SemaphoreType.DMA,
        pltpu.SemaphoreType.DMA,
    ],
)
def _dispatch(x_hbm, pos0_hbm, pos1_hbm, gx_hbm, xbuf, idx0, idx1, sem0,
              sem1):
    wid = lax.axis_index("s") * 2 + lax.axis_index("c")
    base = wid * TPW
    pltpu.sync_copy(x_hbm.at[pl.ds(base, TPW), :], xbuf)
    pltpu.sync_copy(pos0_hbm.at[pl.ds(base, TPW)], idx0)
    pltpu.sync_copy(pos1_hbm.at[pl.ds(base, TPW)], idx1)
    cp0 = pltpu.async_copy(xbuf, gx_hbm.at[idx0], sem0)
    cp1 = pltpu.async_copy(xbuf, gx_hbm.at[idx1], sem1)
    cp0.wait()
    cp1.wait()


def _erf(z):
    return lax.erf(z)


def _gemm_body(be_sref, gx_ref, fc1_ref, fc1b_ref, fc2_ref, fc2b_ref, y_ref):
    del be_sref
    h1 = jnp.dot(gx_ref[...], fc1_ref[0], preferred_element_type=jnp.float32)
    h1 = h1 + fc1b_ref[0]
    a = 0.5 * h1 * (1.0 + _erf(h1 * 0.7071067811865476))
    y = jnp.dot(a.astype(jnp.bfloat16), fc2_ref[0],
                preferred_element_type=jnp.float32)
    y_ref[...] = y + fc2b_ref[0]


def _run_gemm(be40, gx, fc1_w, fc1_b, fc2_w, fc2_b):
    grid_spec = pltpu.PrefetchScalarGridSpec(
        num_scalar_prefetch=1,
        grid=(NBLK,),
        in_specs=[
            pl.BlockSpec((BLK, HD), lambda b, be: (b, 0)),
            pl.BlockSpec((1, HD, FF), lambda b, be: (be[b], 0, 0)),
            pl.BlockSpec((1, 1, FF), lambda b, be: (be[b], 0, 0)),
            pl.BlockSpec((1, FF, HD), lambda b, be: (be[b], 0, 0)),
            pl.BlockSpec((1, 1, HD), lambda b, be: (be[b], 0, 0)),
        ],
        out_specs=pl.BlockSpec((BLK, HD), lambda b, be: (b, 0)),
    )
    return pl.pallas_call(
        _gemm_body,
        grid_spec=grid_spec,
        out_shape=jax.ShapeDtypeStruct((PAD, HD), jnp.float32),
        compiler_params=pltpu.CompilerParams(
            dimension_semantics=("arbitrary",)),
    )(be40, gx, fc1_w.astype(jnp.bfloat16), fc1_b.reshape(NE, 1, FF),
      fc2_w.astype(jnp.bfloat16), fc2_b.reshape(NE, 1, HD))


@functools.partial(
    pl.kernel,
    out_type=jax.ShapeDtypeStruct((T, HD), jnp.float32),
    mesh=_SC_MESH,
    scratch_types=[
        pltpu.VMEM((TPW,), jnp.int32),
        pltpu.VMEM((TPW,), jnp.int32),
        pltpu.VMEM((TPW, HD), jnp.float32),
        pltpu.VMEM((TPW, HD), jnp.float32),
        pltpu.VMEM((TPW, LANES), jnp.float32),
        pltpu.VMEM((TPW, LANES), jnp.float32),
        pltpu.SemaphoreType.DMA,
        pltpu.SemaphoreType.DMA,
    ],
)
def _combine(y_hbm, pos0_hbm, pos1_hbm, w0_hbm, w1_hbm, out_hbm, idx0, idx1,
             rows0, rows1, wv0, wv1, sem0, sem1):
    wid = lax.axis_index("s") * 2 + lax.axis_index("c")
    base = wid * TPW
    pltpu.sync_copy(pos0_hbm.at[pl.ds(base, TPW)], idx0)
    pltpu.sync_copy(pos1_hbm.at[pl.ds(base, TPW)], idx1)
    cp0 = pltpu.async_copy(y_hbm.at[idx0], rows0, sem0)
    cp1 = pltpu.async_copy(y_hbm.at[idx1], rows1, sem1)
    pltpu.sync_copy(w0_hbm.at[pl.ds(base, TPW), :], wv0)
    pltpu.sync_copy(w1_hbm.at[pl.ds(base, TPW), :], wv1)
    cp0.wait()
    cp1.wait()

    def token_body(t, carry):
        a0 = wv0[t, :]
        a1 = wv1[t, :]

        def chunk_body(c, carry2):
            off = pl.multiple_of(c * LANES, LANES)
            r0 = rows0[t, pl.ds(off, LANES)]
            r1 = rows1[t, pl.ds(off, LANES)]
            rows0[t, pl.ds(off, LANES)] = a0 * r0 + a1 * r1
            return carry2

        return lax.fori_loop(0, HD // LANES, chunk_body, carry)

    lax.fori_loop(0, TPW, token_body, 0)
    pltpu.sync_copy(rows0, out_hbm.at[pl.ds(base, TPW), :])


def kernel(x, gate_w, gate_b, fc1_w, fc1_b, fc2_w, fc2_b):
    b, s, h = x.shape
    xf = x.reshape(T, HD)

    pos2, w2, be2, laux2, counts2 = _run_router(xf, gate_w, gate_b)
    pos0 = pos2[:, 0]
    pos1 = pos2[:, 1]
    w0r = jnp.broadcast_to(w2[:, 0:1], (T, LANES))
    w1r = jnp.broadcast_to(w2[:, 1:2], (T, LANES))
    be40 = be2[:, 0]

    xp = lax.bitcast_convert_type(
        xf.astype(jnp.bfloat16).reshape(T, HD // 2, 2), jnp.int32)
    gxp = _dispatch(xp, pos0, pos1)
    gx = lax.bitcast_convert_type(gxp, jnp.bfloat16).reshape(PAD, HD)
    y = _run_gemm(be40, gx, fc1_w, fc1_b, fc2_w, fc2_b)
    out = _combine(y, pos0, pos1, w0r, w1r)

    return out.reshape(b, s, h), laux2[0, 0], counts2[0, :]


# speedup vs baseline: 1.8048x; 1.8048x over previous
import functools

import jax
import jax.numpy as jnp
from jax import lax
from jax.experimental import pallas as pl
from jax.experimental.pallas import tpu as pltpu
from jax.experimental.pallas import tpu_sc as plsc

T = 2048
HD = 768
NE = 8
FF = 4 * HD
BLK = 128
PAD = T * 2 + NE * BLK
NBLK = PAD // BLK
NW = 32
TPW = T // NW
LANES = 16


def _router_body(x_ref, gw_ref, gb_ref, pos_ref, w_ref, be_ref, laux_ref,
                 counts_ref):
    xf = x_ref[...]
    logits = jnp.dot(xf, gw_ref[...],
                     preferred_element_type=jnp.float32) + gb_ref[...]

    idx8 = lax.broadcasted_iota(jnp.int32, (T, NE), 1)
    v0 = jnp.max(logits, axis=1, keepdims=True)
    i0 = jnp.min(jnp.where(logits == v0, idx8, NE), axis=1, keepdims=True)
    masked = jnp.where(idx8 == i0, -jnp.inf, logits)
    v1 = jnp.max(masked, axis=1, keepdims=True)
    i1 = jnp.min(jnp.where(masked == v1, idx8, NE), axis=1, keepdims=True)

    e1 = jnp.exp(v1 - v0)
    denom = 1.0 + e1
    w_ref[:, 0:1] = 1.0 / denom
    w_ref[:, 1:2] = e1 / denom

    p = jnp.exp(logits - v0)
    probs = p / jnp.sum(p, axis=1, keepdims=True)
    pm = jnp.sum(probs, axis=0, keepdims=True) * (1.0 / T)
    laux_ref[...] = jnp.sum(pm * pm, axis=1, keepdims=True) * NE

    oh0 = (idx8 == i0).astype(jnp.float32)
    oh1 = (idx8 == i1).astype(jnp.float32)
    oht = oh0 + oh1

    cum = oht
    sh = 1
    while sh < T:
        cum = cum + jnp.concatenate(
            [jnp.zeros((sh, NE), jnp.float32), cum[:T - sh]], axis=0)
        sh *= 2
    counts = cum[T - 1:T, :]
    counts_ref[...] = counts
    cum_excl = jnp.concatenate(
        [jnp.zeros((1, NE), jnp.float32), cum[:T - 1]], axis=0)

    ci = counts.astype(jnp.int32)
    pc = ((ci + (BLK - 1)) // BLK) * BLK
    oc = pc
    sh = 1
    while sh < NE:
        oc = oc + jnp.concatenate(
            [jnp.zeros((1, sh), jnp.int32), oc[:, :NE - sh]], axis=1)
        sh *= 2
    offs = jnp.concatenate(
        [jnp.zeros((1, 1), jnp.int32), oc[:, :NE - 1]], axis=1)

    row0 = lax.broadcasted_iota(jnp.int32, (NBLK, NE), 0) * BLK
    offs_b = jnp.broadcast_to(offs, (NBLK, NE))
    pc_b = jnp.broadcast_to(pc, (NBLK, NE))
    e_iota = lax.broadcasted_iota(jnp.int32, (NBLK, NE), 1)
    ind = (row0 >= offs_b) & (row0 < offs_b + pc_b)
    be_ref[...] = jnp.sum(jnp.where(ind, e_iota, 0), axis=1, keepdims=True)

    posmat = jnp.broadcast_to(offs.astype(jnp.float32), (T, NE)) + cum_excl
    pos0 = jnp.sum(jnp.where(idx8 == i0, posmat, 0.0), axis=1, keepdims=True)
    pos1 = jnp.sum(jnp.where(idx8 == i1, posmat, 0.0), axis=1, keepdims=True)
    pos_ref[:, 0:1] = pos0.astype(jnp.int32)
    pos_ref[:, 1:2] = pos1.astype(jnp.int32)


def _run_router(xf, gate_w, gate_b):
    return pl.pallas_call(
        _router_body,
        out_shape=[
            jax.ShapeDtypeStruct((T, 2), jnp.int32),
            jax.ShapeDtypeStruct((T, 2), jnp.float32),
            jax.ShapeDtypeStruct((NBLK, 1), jnp.int32),
            jax.ShapeDtypeStruct((1, 1), jnp.float32),
            jax.ShapeDtypeStruct((1, NE), jnp.float32),
        ],
    )(xf, gate_w, gate_b.reshape(1, NE))


_SC_MESH = plsc.VectorSubcoreMesh(core_axis_name="c", subcore_axis_name="s",
                                  num_cores=2, num_subcores=16)


@functools.partial(
    pl.kernel,
    out_type=jax.ShapeDtypeStruct((PAD, HD), jnp.float32),
    mesh=_SC_MESH,
    scratch_types=[
        pltpu.VMEM((TPW, HD), jnp.float32),
        pltpu.VMEM((TPW,), jnp.int32),
        pltpu.VMEM((TPW,), jnp.int32),
        pltpu.SemaphoreType.DMA,
        pltpu.SemaphoreType.DMA,
    ],
)
def _dispatch(x_hbm, pos0_hbm, pos1_hbm, gx_hbm, xbuf, idx0, idx1, sem0,
              sem1):
    wid = lax.axis_index("s") * 2 + lax.axis_index("c")
    base = wid * TPW
    pltpu.sync_copy(x_hbm.at[pl.ds(base, TPW), :], xbuf)
    pltpu.sync_copy(pos0_hbm.at[pl.ds(base, TPW)], idx0)
    pltpu.sync_copy(pos1_hbm.at[pl.ds(base, TPW)], idx1)
    cp0 = pltpu.async_copy(xbuf, gx_hbm.at[idx0], sem0)
    cp1 = pltpu.async_copy(xbuf, gx_hbm.at[idx1], sem1)
    cp0.wait()
    cp1.wait()


def _erf(z):
    return lax.erf(z)


def _gemm_body(be_sref, gx_ref, fc1_ref, fc1b_ref, fc2_ref, fc2b_ref, y_ref):
    del be_sref
    xb = gx_ref[...].astype(jnp.bfloat16)
    h1 = jnp.dot(xb, fc1_ref[0].astype(jnp.bfloat16),
                 preferred_element_type=jnp.float32)
    h1 = h1 + fc1b_ref[0]
    a = 0.5 * h1 * (1.0 + _erf(h1 * 0.7071067811865476))
    y = jnp.dot(a.astype(jnp.bfloat16), fc2_ref[0].astype(jnp.bfloat16),
                preferred_element_type=jnp.float32)
    y_ref[...] = y + fc2b_ref[0]


def _run_gemm(be40, gx, fc1_w, fc1_b, fc2_w, fc2_b):
    grid_spec = pltpu.PrefetchScalarGridSpec(
        num_scalar_prefetch=1,
        grid=(NBLK,),
        in_specs=[
            pl.BlockSpec((BLK, HD), lambda b, be: (b, 0)),
            pl.BlockSpec((1, HD, FF), lambda b, be: (be[b], 0, 0)),
            pl.BlockSpec((1, 1, FF), lambda b, be: (be[b], 0, 0)),
            pl.BlockSpec((1, FF, HD), lambda b, be: (be[b], 0, 0)),
            pl.BlockSpec((1, 1, HD), lambda b, be: (be[b], 0, 0)),
        ],
        out_specs=pl.BlockSpec((BLK, HD), lambda b, be: (b, 0)),
    )
    return pl.pallas_call(
        _gemm_body,
        grid_spec=grid_spec,
        out_shape=jax.ShapeDtypeStruct((PAD, HD), jnp.float32),
        compiler_params=pltpu.CompilerParams(
            dimension_semantics=("arbitrary",)),
    )(be40, gx, fc1_w, fc1_b.reshape(NE, 1, FF),
      fc2_w, fc2_b.reshape(NE, 1, HD))


@functools.partial(
    pl.kernel,
    out_type=jax.ShapeDtypeStruct((T, HD), jnp.float32),
    mesh=_SC_MESH,
    scratch_types=[
        pltpu.VMEM((TPW,), jnp.int32),
        pltpu.VMEM((TPW,), jnp.int32),
        pltpu.VMEM((TPW, HD), jnp.float32),
        pltpu.VMEM((TPW, HD), jnp.float32),
        pltpu.VMEM((TPW, LANES), jnp.float32),
        pltpu.VMEM((TPW, LANES), jnp.float32),
        pltpu.SemaphoreType.DMA,
        pltpu.SemaphoreType.DMA,
    ],
)
def _combine(y_hbm, pos0_hbm, pos1_hbm, w0_hbm, w1_hbm, out_hbm, idx0, idx1,
             rows0, rows1, wv0, wv1, sem0, sem1):
    wid = lax.axis_index("s") * 2 + lax.axis_index("c")
    base = wid * TPW
    pltpu.sync_copy(pos0_hbm.at[pl.ds(base, TPW)], idx0)
    pltpu.sync_copy(pos1_hbm.at[pl.ds(base, TPW)], idx1)
    cp0 = pltpu.async_copy(y_hbm.at[idx0], rows0, sem0)
    cp1 = pltpu.async_copy(y_hbm.at[idx1], rows1, sem1)
    pltpu.sync_copy(w0_hbm.at[pl.ds(base, TPW), :], wv0)
    pltpu.sync_copy(w1_hbm.at[pl.ds(base, TPW), :], wv1)
    cp0.wait()
    cp1.wait()

    def token_body(t, carry):
        a0 = wv0[t, :]
        a1 = wv1[t, :]

        def chunk_body(c, carry2):
            off = pl.multiple_of(c * LANES, LANES)
            r0 = rows0[t, pl.ds(off, LANES)]
            r1 = rows1[t, pl.ds(off, LANES)]
            rows0[t, pl.ds(off, LANES)] = a0 * r0 + a1 * r1
            return carry2

        return lax.fori_loop(0, HD // LANES, chunk_body, carry)

    lax.fori_loop(0, TPW, token_body, 0)
    pltpu.sync_copy(rows0, out_hbm.at[pl.ds(base, TPW), :])


def kernel(x, gate_w, gate_b, fc1_w, fc1_b, fc2_w, fc2_b):
    b, s, h = x.shape
    xf = x.reshape(T, HD)

    pos2, w2, be2, laux2, counts2 = _run_router(xf, gate_w, gate_b)
    pos0 = pos2[:, 0]
    pos1 = pos2[:, 1]
    w0r = jnp.broadcast_to(w2[:, 0:1], (T, LANES))
    w1r = jnp.broadcast_to(w2[:, 1:2], (T, LANES))
    be40 = be2[:, 0]

    gx = _dispatch(xf, pos0, pos1)
    y = _run_gemm(be40, gx, fc1_w, fc1_b, fc2_w, fc2_b)
    out = _combine(y, pos0, pos1, w0r, w1r)

    return out.reshape(b, s, h), laux2[0, 0], counts2[0, :]


# speedup vs baseline: 1.9435x; 1.0768x over previous
import functools

import jax
import jax.numpy as jnp
from jax import lax
from jax.experimental import pallas as pl
from jax.experimental.pallas import tpu as pltpu
from jax.experimental.pallas import tpu_sc as plsc

T = 2048
HD = 768
NE = 8
FF = 4 * HD
BLK = 128
PAD = T * 2 + NE * BLK
NBLK = PAD // BLK
NW = 32
TPW = T // NW
LANES = 16


def _router_body(x_ref, gw_ref, gb_ref, pos_ref, w_ref, be_ref, laux_ref,
                 counts_ref):
    xf = x_ref[...]
    logits = jnp.dot(xf, gw_ref[...],
                     preferred_element_type=jnp.float32) + gb_ref[...]

    idx8 = lax.broadcasted_iota(jnp.int32, (T, NE), 1)
    v0 = jnp.max(logits, axis=1, keepdims=True)
    i0 = jnp.min(jnp.where(logits == v0, idx8, NE), axis=1, keepdims=True)
    masked = jnp.where(idx8 == i0, -jnp.inf, logits)
    v1 = jnp.max(masked, axis=1, keepdims=True)
    i1 = jnp.min(jnp.where(masked == v1, idx8, NE), axis=1, keepdims=True)

    e1 = jnp.exp(v1 - v0)
    denom = 1.0 + e1
    w_ref[:, 0:1] = 1.0 / denom
    w_ref[:, 1:2] = e1 / denom

    p = jnp.exp(logits - v0)
    probs = p / jnp.sum(p, axis=1, keepdims=True)
    pm = jnp.sum(probs, axis=0, keepdims=True) * (1.0 / T)
    laux_ref[...] = jnp.sum(pm * pm, axis=1, keepdims=True) * NE

    oh0 = (idx8 == i0).astype(jnp.float32)
    oh1 = (idx8 == i1).astype(jnp.float32)
    oht = oh0 + oh1

    cum = oht
    sh = 1
    while sh < T:
        cum = cum + jnp.concatenate(
            [jnp.zeros((sh, NE), jnp.float32), cum[:T - sh]], axis=0)
        sh *= 2
    counts = cum[T - 1:T, :]
    counts_ref[...] = counts
    cum_excl = jnp.concatenate(
        [jnp.zeros((1, NE), jnp.float32), cum[:T - 1]], axis=0)

    ci = counts.astype(jnp.int32)
    pc = ((ci + (BLK - 1)) // BLK) * BLK
    oc = pc
    sh = 1
    while sh < NE:
        oc = oc + jnp.concatenate(
            [jnp.zeros((1, sh), jnp.int32), oc[:, :NE - sh]], axis=1)
        sh *= 2
    offs = jnp.concatenate(
        [jnp.zeros((1, 1), jnp.int32), oc[:, :NE - 1]], axis=1)

    row0 = lax.broadcasted_iota(jnp.int32, (NBLK, NE), 0) * BLK
    offs_b = jnp.broadcast_to(offs, (NBLK, NE))
    pc_b = jnp.broadcast_to(pc, (NBLK, NE))
    e_iota = lax.broadcasted_iota(jnp.int32, (NBLK, NE), 1)
    ind = (row0 >= offs_b) & (row0 < offs_b + pc_b)
    be_ref[...] = jnp.sum(jnp.where(ind, e_iota, 0), axis=1, keepdims=True)

    posmat = jnp.broadcast_to(offs.astype(jnp.float32), (T, NE)) + cum_excl
    pos0 = jnp.sum(jnp.where(idx8 == i0, posmat, 0.0), axis=1, keepdims=True)
    pos1 = jnp.sum(jnp.where(idx8 == i1, posmat, 0.0), axis=1, keepdims=True)
    pos_ref[:, 0:1] = pos0.astype(jnp.int32)
    pos_ref[:, 1:2] = pos1.astype(jnp.int32)


def _run_router(xf, gate_w, gate_b):
    return pl.pallas_call(
        _router_body,
        out_shape=[
            jax.ShapeDtypeStruct((T, 2), jnp.int32),
            jax.ShapeDtypeStruct((T, 2), jnp.float32),
            jax.ShapeDtypeStruct((NBLK, 1), jnp.int32),
            jax.ShapeDtypeStruct((1, 1), jnp.float32),
            jax.ShapeDtypeStruct((1, NE), jnp.float32),
        ],
    )(xf, gate_w, gate_b.reshape(1, NE))


_SC_MESH = plsc.VectorSubcoreMesh(core_axis_name="c", subcore_axis_name="s",
                                  num_cores=2, num_subcores=16)


@functools.partial(
    pl.kernel,
    out_type=jax.ShapeDtypeStruct((PAD, HD), jnp.float32),
    mesh=_SC_MESH,
    scratch_types=[
        pltpu.VMEM((TPW, HD), jnp.float32),
        pltpu.VMEM((TPW,), jnp.int32),
        pltpu.VMEM((TPW,), jnp.int32),
        pltpu.SemaphoreType.DMA,
        pltpu.SemaphoreType.DMA,
        pltpu.SemaphoreType.DMA,
    ],
)
def _dispatch(x_hbm, pos0_hbm, pos1_hbm, gx_hbm, xbuf, idx0, idx1, sem0,
              sem1, sem2):
    wid = lax.axis_index("s") * 2 + lax.axis_index("c")
    base = wid * TPW
    cpx = pltpu.async_copy(x_hbm.at[pl.ds(base, TPW), :], xbuf, sem0)
    cpi0 = pltpu.async_copy(pos0_hbm.at[pl.ds(base, TPW)], idx0, sem1)
    cpi1 = pltpu.async_copy(pos1_hbm.at[pl.ds(base, TPW)], idx1, sem2)
    cpi0.wait()
    cpi1.wait()
    cpx.wait()
    cp0 = pltpu.async_copy(xbuf, gx_hbm.at[idx0], sem1)
    cp1 = pltpu.async_copy(xbuf, gx_hbm.at[idx1], sem2)
    cp0.wait()
    cp1.wait()


def _erf(z):
    return lax.erf(z)


def _gemm_body(be_sref, gx_ref, fc1_ref, fc1b_ref, fc2_ref, fc2b_ref, y_ref):
    del be_sref
    xb = gx_ref[...].astype(jnp.bfloat16)
    h1 = jnp.dot(xb, fc1_ref[0].astype(jnp.bfloat16),
                 preferred_element_type=jnp.float32)
    h1 = h1 + fc1b_ref[0]
    a = 0.5 * h1 * (1.0 + _erf(h1 * 0.7071067811865476))
    y = jnp.dot(a.astype(jnp.bfloat16), fc2_ref[0].astype(jnp.bfloat16),
                preferred_element_type=jnp.float32)
    y_ref[...] = y + fc2b_ref[0]


def _run_gemm(be40, gx, fc1_w, fc1_b, fc2_w, fc2_b):
    grid_spec = pltpu.PrefetchScalarGridSpec(
        num_scalar_prefetch=1,
        grid=(NBLK,),
        in_specs=[
            pl.BlockSpec((BLK, HD), lambda b, be: (b, 0)),
            pl.BlockSpec((1, HD, FF), lambda b, be: (be[b], 0, 0)),
            pl.BlockSpec((1, 1, FF), lambda b, be: (be[b], 0, 0)),
            pl.BlockSpec((1, FF, HD), lambda b, be: (be[b], 0, 0)),
            pl.BlockSpec((1, 1, HD), lambda b, be: (be[b], 0, 0)),
        ],
        out_specs=pl.BlockSpec((BLK, HD), lambda b, be: (b, 0)),
    )
    return pl.pallas_call(
        _gemm_body,
        grid_spec=grid_spec,
        out_shape=jax.ShapeDtypeStruct((PAD, HD), jnp.float32),
        compiler_params=pltpu.CompilerParams(
            dimension_semantics=("arbitrary",)),
    )(be40, gx, fc1_w, fc1_b.reshape(NE, 1, FF),
      fc2_w, fc2_b.reshape(NE, 1, HD))


@functools.partial(
    pl.kernel,
    out_type=jax.ShapeDtypeStruct((T, HD), jnp.float32),
    mesh=_SC_MESH,
    scratch_types=[
        pltpu.VMEM((TPW,), jnp.int32),
        pltpu.VMEM((TPW,), jnp.int32),
        pltpu.VMEM((TPW, HD), jnp.float32),
        pltpu.VMEM((TPW, HD), jnp.float32),
        pltpu.VMEM((TPW, LANES), jnp.float32),
        pltpu.VMEM((TPW, LANES), jnp.float32),
        pltpu.SemaphoreType.DMA,
        pltpu.SemaphoreType.DMA,
    ],
)
def _combine(y_hbm, pos0_hbm, pos1_hbm, w0_hbm, w1_hbm, out_hbm, idx0, idx1,
             rows0, rows1, wv0, wv1, sem0, sem1):
    wid = lax.axis_index("s") * 2 + lax.axis_index("c")
    base = wid * TPW
    pltpu.sync_copy(pos0_hbm.at[pl.ds(base, TPW)], idx0)
    pltpu.sync_copy(pos1_hbm.at[pl.ds(base, TPW)], idx1)
    cp0 = pltpu.async_copy(y_hbm.at[idx0], rows0, sem0)
    cp1 = pltpu.async_copy(y_hbm.at[idx1], rows1, sem1)
    pltpu.sync_copy(w0_hbm.at[pl.ds(base, TPW), :], wv0)
    pltpu.sync_copy(w1_hbm.at[pl.ds(base, TPW), :], wv1)
    cp0.wait()
    cp1.wait()

    def token_body(t, carry):
        a0 = wv0[t, :]
        a1 = wv1[t, :]
        for c in range(HD // LANES):
            r0 = rows0[t, pl.ds(c * LANES, LANES)]
            r1 = rows1[t, pl.ds(c * LANES, LANES)]
            rows0[t, pl.ds(c * LANES, LANES)] = a0 * r0 + a1 * r1
        return carry

    lax.fori_loop(0, TPW, token_body, 0)
    pltpu.sync_copy(rows0, out_hbm.at[pl.ds(base, TPW), :])


def kernel(x, gate_w, gate_b, fc1_w, fc1_b, fc2_w, fc2_b):
    b, s, h = x.shape
    xf = x.reshape(T, HD)

    pos2, w2, be2, laux2, counts2 = _run_router(xf, gate_w, gate_b)
    pos0 = pos2[:, 0]
    pos1 = pos2[:, 1]
    w0r = jnp.broadcast_to(w2[:, 0:1], (T, LANES))
    w1r = jnp.broadcast_to(w2[:, 1:2], (T, LANES))
    be40 = be2[:, 0]

    gx = _dispatch(xf, pos0, pos1)
    y = _run_gemm(be40, gx, fc1_w, fc1_b, fc2_w, fc2_b)
    out = _combine(y, pos0, pos1, w0r, w1r)

    return out.reshape(b, s, h), laux2[0, 0], counts2[0, :]


# speedup vs baseline: 1.9553x; 1.0061x over previous
import functools

import jax
import jax.numpy as jnp
from jax import lax
from jax.experimental import pallas as pl
from jax.experimental.pallas import tpu as pltpu
from jax.experimental.pallas import tpu_sc as plsc

T = 2048
HD = 768
NE = 8
FF = 4 * HD
BLK = 128
PAD = T * 2 + NE * BLK
NBLK = PAD // BLK
NW = 32
TPW = T // NW
LANES = 16


def _router_body(x_ref, gw_ref, gb_ref, pos0_ref, pos1_ref, w0_ref, w1_ref,
                 be_ref, laux_ref, counts_ref):
    xf = x_ref[...]
    logits = jnp.dot(xf, gw_ref[...],
                     preferred_element_type=jnp.float32) + gb_ref[...]

    idx8 = lax.broadcasted_iota(jnp.int32, (T, NE), 1)
    v0 = jnp.max(logits, axis=1, keepdims=True)
    i0 = jnp.min(jnp.where(logits == v0, idx8, NE), axis=1, keepdims=True)
    masked = jnp.where(idx8 == i0, -jnp.inf, logits)
    v1 = jnp.max(masked, axis=1, keepdims=True)
    i1 = jnp.min(jnp.where(masked == v1, idx8, NE), axis=1, keepdims=True)

    e1 = jnp.exp(v1 - v0)
    denom = 1.0 + e1
    w0_ref[...] = jnp.broadcast_to(1.0 / denom, (T, LANES))
    w1_ref[...] = jnp.broadcast_to(e1 / denom, (T, LANES))

    p = jnp.exp(logits - v0)
    probs = p / jnp.sum(p, axis=1, keepdims=True)
    pm = jnp.sum(probs, axis=0, keepdims=True) * (1.0 / T)
    laux_ref[...] = jnp.sum(pm * pm, axis=1, keepdims=True) * NE

    oh0 = (idx8 == i0).astype(jnp.float32)
    oh1 = (idx8 == i1).astype(jnp.float32)
    oht = oh0 + oh1

    cum = oht
    sh = 1
    while sh < T:
        cum = cum + jnp.concatenate(
            [jnp.zeros((sh, NE), jnp.float32), cum[:T - sh]], axis=0)
        sh *= 2
    counts = cum[T - 1:T, :]
    counts_ref[...] = counts
    cum_excl = jnp.concatenate(
        [jnp.zeros((1, NE), jnp.float32), cum[:T - 1]], axis=0)

    ci = counts.astype(jnp.int32)
    pc = ((ci + (BLK - 1)) // BLK) * BLK
    oc = pc
    sh = 1
    while sh < NE:
        oc = oc + jnp.concatenate(
            [jnp.zeros((1, sh), jnp.int32), oc[:, :NE - sh]], axis=1)
        sh *= 2
    offs = jnp.concatenate(
        [jnp.zeros((1, 1), jnp.int32), oc[:, :NE - 1]], axis=1)

    row0 = lax.broadcasted_iota(jnp.int32, (NBLK, NE), 0) * BLK
    offs_b = jnp.broadcast_to(offs, (NBLK, NE))
    pc_b = jnp.broadcast_to(pc, (NBLK, NE))
    e_iota = lax.broadcasted_iota(jnp.int32, (NBLK, NE), 1)
    ind = (row0 >= offs_b) & (row0 < offs_b + pc_b)
    be_ref[...] = jnp.sum(jnp.where(ind, e_iota, 0), axis=1, keepdims=True)

    posmat = jnp.broadcast_to(offs.astype(jnp.float32), (T, NE)) + cum_excl
    pos0 = jnp.sum(jnp.where(idx8 == i0, posmat, 0.0), axis=1, keepdims=True)
    pos1 = jnp.sum(jnp.where(idx8 == i1, posmat, 0.0), axis=1, keepdims=True)
    pos0_ref[...] = pos0.astype(jnp.int32)
    pos1_ref[...] = pos1.astype(jnp.int32)


def _run_router(xf, gate_w, gate_b):
    return pl.pallas_call(
        _router_body,
        out_shape=[
            jax.ShapeDtypeStruct((T, 1), jnp.int32),
            jax.ShapeDtypeStruct((T, 1), jnp.int32),
            jax.ShapeDtypeStruct((T, LANES), jnp.float32),
            jax.ShapeDtypeStruct((T, LANES), jnp.float32),
            jax.ShapeDtypeStruct((NBLK, 1), jnp.int32),
            jax.ShapeDtypeStruct((1, 1), jnp.float32),
            jax.ShapeDtypeStruct((1, NE), jnp.float32),
        ],
    )(xf, gate_w, gate_b.reshape(1, NE))


_SC_MESH = plsc.VectorSubcoreMesh(core_axis_name="c", subcore_axis_name="s",
                                  num_cores=2, num_subcores=16)


@functools.partial(
    pl.kernel,
    out_type=jax.ShapeDtypeStruct((PAD, HD), jnp.float32),
    mesh=_SC_MESH,
    scratch_types=[
        pltpu.VMEM((TPW, HD), jnp.float32),
        pltpu.VMEM((TPW,), jnp.int32),
        pltpu.VMEM((TPW,), jnp.int32),
        pltpu.SemaphoreType.DMA,
        pltpu.SemaphoreType.DMA,
        pltpu.SemaphoreType.DMA,
    ],
)
def _dispatch(x_hbm, pos0_hbm, pos1_hbm, gx_hbm, xbuf, idx0, idx1, sem0,
              sem1, sem2):
    wid = lax.axis_index("s") * 2 + lax.axis_index("c")
    base = wid * TPW
    cpx = pltpu.async_copy(x_hbm.at[pl.ds(base, TPW), :], xbuf, sem0)
    cpi0 = pltpu.async_copy(pos0_hbm.at[pl.ds(base, TPW)], idx0, sem1)
    cpi1 = pltpu.async_copy(pos1_hbm.at[pl.ds(base, TPW)], idx1, sem2)
    cpi0.wait()
    cpi1.wait()
    cpx.wait()
    cp0 = pltpu.async_copy(xbuf, gx_hbm.at[idx0], sem1)
    cp1 = pltpu.async_copy(xbuf, gx_hbm.at[idx1], sem2)
    cp0.wait()
    cp1.wait()


def _erf(z):
    return lax.erf(z)


def _gemm_body(be_sref, gx_ref, fc1_ref, fc1b_ref, fc2_ref, fc2b_ref, y_ref):
    del be_sref
    xb = gx_ref[...].astype(jnp.bfloat16)
    h1 = jnp.dot(xb, fc1_ref[0].astype(jnp.bfloat16),
                 preferred_element_type=jnp.float32)
    h1 = h1 + fc1b_ref[0]
    a = 0.5 * h1 * (1.0 + _erf(h1 * 0.7071067811865476))
    y = jnp.dot(a.astype(jnp.bfloat16), fc2_ref[0].astype(jnp.bfloat16),
                preferred_element_type=jnp.float32)
    y_ref[...] = y + fc2b_ref[0]


def _run_gemm(be40, gx, fc1_w, fc1_b, fc2_w, fc2_b):
    grid_spec = pltpu.PrefetchScalarGridSpec(
        num_scalar_prefetch=1,
        grid=(NBLK,),
        in_specs=[
            pl.BlockSpec((BLK, HD), lambda b, be: (b, 0)),
            pl.BlockSpec((1, HD, FF), lambda b, be: (be[b], 0, 0)),
            pl.BlockSpec((1, 1, FF), lambda b, be: (be[b], 0, 0)),
            pl.BlockSpec((1, FF, HD), lambda b, be: (be[b], 0, 0)),
            pl.BlockSpec((1, 1, HD), lambda b, be: (be[b], 0, 0)),
        ],
        out_specs=pl.BlockSpec((BLK, HD), lambda b, be: (b, 0)),
    )
    return pl.pallas_call(
        _gemm_body,
        grid_spec=grid_spec,
        out_shape=jax.ShapeDtypeStruct((PAD, HD), jnp.float32),
        compiler_params=pltpu.CompilerParams(
            dimension_semantics=("arbitrary",)),
    )(be40, gx, fc1_w, fc1_b.reshape(NE, 1, FF),
      fc2_w, fc2_b.reshape(NE, 1, HD))


@functools.partial(
    pl.kernel,
    out_type=jax.ShapeDtypeStruct((T, HD), jnp.float32),
    mesh=_SC_MESH,
    scratch_types=[
        pltpu.VMEM((TPW,), jnp.int32),
        pltpu.VMEM((TPW,), jnp.int32),
        pltpu.VMEM((TPW, HD), jnp.float32),
        pltpu.VMEM((TPW, HD), jnp.float32),
        pltpu.VMEM((TPW, LANES), jnp.float32),
        pltpu.VMEM((TPW, LANES), jnp.float32),
        pltpu.SemaphoreType.DMA,
        pltpu.SemaphoreType.DMA,
    ],
)
def _combine(y_hbm, pos0_hbm, pos1_hbm, w0_hbm, w1_hbm, out_hbm, idx0, idx1,
             rows0, rows1, wv0, wv1, sem0, sem1):
    wid = lax.axis_index("s") * 2 + lax.axis_index("c")
    base = wid * TPW
    pltpu.sync_copy(pos0_hbm.at[pl.ds(base, TPW)], idx0)
    pltpu.sync_copy(pos1_hbm.at[pl.ds(base, TPW)], idx1)
    cp0 = pltpu.async_copy(y_hbm.at[idx0], rows0, sem0)
    cp1 = pltpu.async_copy(y_hbm.at[idx1], rows1, sem1)
    pltpu.sync_copy(w0_hbm.at[pl.ds(base, TPW), :], wv0)
    pltpu.sync_copy(w1_hbm.at[pl.ds(base, TPW), :], wv1)
    cp0.wait()
    cp1.wait()

    def token_body(t, carry):
        a0 = wv0[t, :]
        a1 = wv1[t, :]
        for c in range(HD // LANES):
            r0 = rows0[t, pl.ds(c * LANES, LANES)]
            r1 = rows1[t, pl.ds(c * LANES, LANES)]
            rows0[t, pl.ds(c * LANES, LANES)] = a0 * r0 + a1 * r1
        return carry

    lax.fori_loop(0, TPW, token_body, 0)
    pltpu.sync_copy(rows0, out_hbm.at[pl.ds(base, TPW), :])


def kernel(x, gate_w, gate_b, fc1_w, fc1_b, fc2_w, fc2_b):
    b, s, h = x.shape
    xf = x.reshape(T, HD)

    p0, p1, w0r, w1r, be2, laux2, counts2 = _run_router(xf, gate_w, gate_b)
    pos0 = p0.reshape(T)
    pos1 = p1.reshape(T)
    be40 = be2.reshape(NBLK)

    gx = _dispatch(xf, pos0, pos1)
    y = _run_gemm(be40, gx, fc1_w, fc1_b, fc2_w, fc2_b)
    out = _combine(y, pos0, pos1, w0r, w1r)

    return out.reshape(b, s, h), laux2.reshape(()), counts2.reshape(NE)


# speedup vs baseline: 2.1704x; 1.1100x over previous
import functools

import jax
import jax.numpy as jnp
from jax import lax
from jax.experimental import pallas as pl
from jax.experimental.pallas import tpu as pltpu
from jax.experimental.pallas import tpu_sc as plsc

T = 2048
HD = 768
NE = 8
FF = 4 * HD
BLK = 128
PAD = T * 2 + NE * BLK
NBLK = PAD // BLK
NW = 32
TPW = T // NW
LANES = 16


def _router_body(x_ref, gw_ref, gb_ref, pos0_ref, pos1_ref, w0_ref, w1_ref,
                 be_ref, isst_ref, slot_ref, nxe_ref, hasnx_ref, laux_ref,
                 counts_ref):
    xf = x_ref[...]
    logits = jnp.dot(xf, gw_ref[...],
                     preferred_element_type=jnp.float32) + gb_ref[...]

    idx8 = lax.broadcasted_iota(jnp.int32, (T, NE), 1)
    v0 = jnp.max(logits, axis=1, keepdims=True)
    i0 = jnp.min(jnp.where(logits == v0, idx8, NE), axis=1, keepdims=True)
    masked = jnp.where(idx8 == i0, -jnp.inf, logits)
    v1 = jnp.max(masked, axis=1, keepdims=True)
    i1 = jnp.min(jnp.where(masked == v1, idx8, NE), axis=1, keepdims=True)

    e1 = jnp.exp(v1 - v0)
    denom = 1.0 + e1
    w0_ref[...] = jnp.broadcast_to(1.0 / denom, (T, LANES))
    w1_ref[...] = jnp.broadcast_to(e1 / denom, (T, LANES))

    p = jnp.exp(logits - v0)
    probs = p / jnp.sum(p, axis=1, keepdims=True)
    pm = jnp.sum(probs, axis=0, keepdims=True) * (1.0 / T)
    laux_ref[...] = jnp.sum(pm * pm, axis=1, keepdims=True) * NE

    oh0 = (idx8 == i0).astype(jnp.float32)
    oh1 = (idx8 == i1).astype(jnp.float32)
    oht = oh0 + oh1

    cum = oht
    sh = 1
    while sh < T:
        cum = cum + jnp.concatenate(
            [jnp.zeros((sh, NE), jnp.float32), cum[:T - sh]], axis=0)
        sh *= 2
    counts = cum[T - 1:T, :]
    counts_ref[...] = counts
    cum_excl = jnp.concatenate(
        [jnp.zeros((1, NE), jnp.float32), cum[:T - 1]], axis=0)

    ci = counts.astype(jnp.int32)
    pc = ((ci + (BLK - 1)) // BLK) * BLK
    oc = pc
    sh = 1
    while sh < NE:
        oc = oc + jnp.concatenate(
            [jnp.zeros((1, sh), jnp.int32), oc[:, :NE - sh]], axis=1)
        sh *= 2
    offs = jnp.concatenate(
        [jnp.zeros((1, 1), jnp.int32), oc[:, :NE - 1]], axis=1)

    row0 = lax.broadcasted_iota(jnp.int32, (NBLK, NE), 0) * BLK
    offs_b = jnp.broadcast_to(offs, (NBLK, NE))
    pc_b = jnp.broadcast_to(pc, (NBLK, NE))
    e_iota = lax.broadcasted_iota(jnp.int32, (NBLK, NE), 1)
    done = (row0 >= offs_b + pc_b).astype(jnp.int32)
    be_col = jnp.minimum(jnp.sum(done, axis=1, keepdims=True), NE - 1)
    be_ref[...] = be_col
    prev = jnp.concatenate(
        [jnp.full((1, 1), -1, jnp.int32), be_col[:NBLK - 1]], axis=0)
    isst_ref[...] = (be_col != prev).astype(jnp.int32)
    be_b = jnp.broadcast_to(be_col, (NBLK, NE))
    present = pc_b > 0
    slot_ref[...] = jnp.sum(
        jnp.where((e_iota < be_b) & present, 1, 0), axis=1, keepdims=True) % 2
    nx = jnp.min(jnp.where((e_iota > be_b) & present, e_iota, NE), axis=1,
                 keepdims=True)
    hasnx_ref[...] = (nx < NE).astype(jnp.int32)
    nxe_ref[...] = jnp.minimum(nx, NE - 1)

    posmat = jnp.broadcast_to(offs.astype(jnp.float32), (T, NE)) + cum_excl
    pos0 = jnp.sum(jnp.where(idx8 == i0, posmat, 0.0), axis=1, keepdims=True)
    pos1 = jnp.sum(jnp.where(idx8 == i1, posmat, 0.0), axis=1, keepdims=True)
    pos0_ref[...] = pos0.astype(jnp.int32)
    pos1_ref[...] = pos1.astype(jnp.int32)


def _run_router(xf, gate_w, gate_b):
    return pl.pallas_call(
        _router_body,
        out_shape=[
            jax.ShapeDtypeStruct((T, 1), jnp.int32),
            jax.ShapeDtypeStruct((T, 1), jnp.int32),
            jax.ShapeDtypeStruct((T, LANES), jnp.float32),
            jax.ShapeDtypeStruct((T, LANES), jnp.float32),
            jax.ShapeDtypeStruct((NBLK, 1), jnp.int32),
            jax.ShapeDtypeStruct((NBLK, 1), jnp.int32),
            jax.ShapeDtypeStruct((NBLK, 1), jnp.int32),
            jax.ShapeDtypeStruct((NBLK, 1), jnp.int32),
            jax.ShapeDtypeStruct((NBLK, 1), jnp.int32),
            jax.ShapeDtypeStruct((1, 1), jnp.float32),
            jax.ShapeDtypeStruct((1, NE), jnp.float32),
        ],
    )(xf, gate_w, gate_b.reshape(1, NE))


_SC_MESH = plsc.VectorSubcoreMesh(core_axis_name="c", subcore_axis_name="s",
                                  num_cores=2, num_subcores=16)


@functools.partial(
    pl.kernel,
    out_type=jax.ShapeDtypeStruct((PAD, HD), jnp.float32),
    mesh=_SC_MESH,
    scratch_types=[
        pltpu.VMEM((TPW, HD), jnp.float32),
        pltpu.VMEM((TPW,), jnp.int32),
        pltpu.VMEM((TPW,), jnp.int32),
        pltpu.SemaphoreType.DMA,
        pltpu.SemaphoreType.DMA,
        pltpu.SemaphoreType.DMA,
    ],
)
def _dispatch(x_hbm, pos0_hbm, pos1_hbm, gx_hbm, xbuf, idx0, idx1, sem0,
              sem1, sem2):
    wid = lax.axis_index("s") * 2 + lax.axis_index("c")
    base = wid * TPW
    cpx = pltpu.async_copy(x_hbm.at[pl.ds(base, TPW), :], xbuf, sem0)
    cpi0 = pltpu.async_copy(pos0_hbm.at[pl.ds(base, TPW)], idx0, sem1)
    cpi1 = pltpu.async_copy(pos1_hbm.at[pl.ds(base, TPW)], idx1, sem2)
    cpi0.wait()
    cpi1.wait()
    cpx.wait()
    cp0 = pltpu.async_copy(xbuf, gx_hbm.at[idx0], sem1)
    cp1 = pltpu.async_copy(xbuf, gx_hbm.at[idx1], sem2)
    cp0.wait()
    cp1.wait()


def _erf(z):
    return lax.erf(z)


def _gemm_body(be_s, isst_s, slot_s, nxe_s, hasnx_s,
               gx_ref, fc1_hbm, fc1b_ref, fc2_hbm, fc2b_ref, y_ref,
               f1buf, f2buf, sems):
    b = pl.program_id(0)
    slot = slot_s[b]

    @pl.when(b == 0)
    def _prologue():
        pltpu.make_async_copy(fc1_hbm.at[be_s[0]], f1buf.at[0],
                              sems.at[0, 0]).start()
        pltpu.make_async_copy(fc2_hbm.at[be_s[0]], f2buf.at[0],
                              sems.at[1, 0]).start()

    @pl.when(isst_s[b] == 1)
    def _wait_weights():
        pltpu.make_async_copy(fc1_hbm.at[be_s[b]], f1buf.at[slot],
                              sems.at[0, slot]).wait()
        pltpu.make_async_copy(fc2_hbm.at[be_s[b]], f2buf.at[slot],
                              sems.at[1, slot]).wait()

    @pl.when((isst_s[b] == 1) & (hasnx_s[b] == 1))
    def _prefetch_next_run():
        nslot = 1 - slot
        pltpu.make_async_copy(fc1_hbm.at[nxe_s[b]], f1buf.at[nslot],
                              sems.at[0, nslot]).start()
        pltpu.make_async_copy(fc2_hbm.at[nxe_s[b]], f2buf.at[nslot],
                              sems.at[1, nslot]).start()

    xb = gx_ref[...].astype(jnp.bfloat16)
    h1 = jnp.dot(xb, f1buf[slot].astype(jnp.bfloat16),
                 preferred_element_type=jnp.float32)
    h1 = h1 + fc1b_ref[0]
    a = 0.5 * h1 * (1.0 + _erf(h1 * 0.7071067811865476))
    y = jnp.dot(a.astype(jnp.bfloat16), f2buf[slot].astype(jnp.bfloat16),
                preferred_element_type=jnp.float32)
    y_ref[...] = y + fc2b_ref[0]


def _run_gemm(be40, isst40, slot40, nxe40, hasnx40, gx, fc1_w, fc1_b, fc2_w,
              fc2_b):
    grid_spec = pltpu.PrefetchScalarGridSpec(
        num_scalar_prefetch=5,
        grid=(NBLK,),
        in_specs=[
            pl.BlockSpec((BLK, HD), lambda b, *_: (b, 0)),
            pl.BlockSpec(memory_space=pltpu.MemorySpace.HBM),
            pl.BlockSpec((1, 1, FF), lambda b, be, *_: (be[b], 0, 0)),
            pl.BlockSpec(memory_space=pltpu.MemorySpace.HBM),
            pl.BlockSpec((1, 1, HD), lambda b, be, *_: (be[b], 0, 0)),
        ],
        out_specs=pl.BlockSpec((BLK, HD), lambda b, *_: (b, 0)),
        scratch_shapes=[
            pltpu.VMEM((2, HD, FF), jnp.float32),
            pltpu.VMEM((2, FF, HD), jnp.float32),
            pltpu.SemaphoreType.DMA((2, 2)),
        ],
    )
    return pl.pallas_call(
        _gemm_body,
        grid_spec=grid_spec,
        out_shape=jax.ShapeDtypeStruct((PAD, HD), jnp.float32),
        compiler_params=pltpu.CompilerParams(
            dimension_semantics=("arbitrary",)),
    )(be40, isst40, slot40, nxe40, hasnx40, gx, fc1_w,
      fc1_b.reshape(NE, 1, FF), fc2_w, fc2_b.reshape(NE, 1, HD))


@functools.partial(
    pl.kernel,
    out_type=jax.ShapeDtypeStruct((T, HD), jnp.float32),
    mesh=_SC_MESH,
    scratch_types=[
        pltpu.VMEM((TPW,), jnp.int32),
        pltpu.VMEM((TPW,), jnp.int32),
        pltpu.VMEM((TPW, HD), jnp.float32),
        pltpu.VMEM((TPW, HD), jnp.float32),
        pltpu.VMEM((TPW, LANES), jnp.float32),
        pltpu.VMEM((TPW, LANES), jnp.float32),
        pltpu.SemaphoreType.DMA,
        pltpu.SemaphoreType.DMA,
    ],
)
def _combine(y_hbm, pos0_hbm, pos1_hbm, w0_hbm, w1_hbm, out_hbm, idx0, idx1,
             rows0, rows1, wv0, wv1, sem0, sem1):
    wid = lax.axis_index("s") * 2 + lax.axis_index("c")
    base = wid * TPW
    pltpu.sync_copy(pos0_hbm.at[pl.ds(base, TPW)], idx0)
    pltpu.sync_copy(pos1_hbm.at[pl.ds(base, TPW)], idx1)
    cp0 = pltpu.async_copy(y_hbm.at[idx0], rows0, sem0)
    cp1 = pltpu.async_copy(y_hbm.at[idx1], rows1, sem1)
    pltpu.sync_copy(w0_hbm.at[pl.ds(base, TPW), :], wv0)
    pltpu.sync_copy(w1_hbm.at[pl.ds(base, TPW), :], wv1)
    cp0.wait()
    cp1.wait()

    def token_body(t, carry):
        a0 = wv0[t, :]
        a1 = wv1[t, :]
        for c in range(HD // LANES):
            r0 = rows0[t, pl.ds(c * LANES, LANES)]
            r1 = rows1[t, pl.ds(c * LANES, LANES)]
            rows0[t, pl.ds(c * LANES, LANES)] = a0 * r0 + a1 * r1
        return carry

    lax.fori_loop(0, TPW, token_body, 0)
    pltpu.sync_copy(rows0, out_hbm.at[pl.ds(base, TPW), :])


def kernel(x, gate_w, gate_b, fc1_w, fc1_b, fc2_w, fc2_b):
    b, s, h = x.shape
    xf = x.reshape(T, HD)

    (p0, p1, w0r, w1r, be2, isst2, slot2, nxe2, hasnx2, laux2,
     counts2) = _run_router(xf, gate_w, gate_b)
    pos0 = p0.reshape(T)
    pos1 = p1.reshape(T)

    gx = _dispatch(xf, pos0, pos1)
    y = _run_gemm(be2.reshape(NBLK), isst2.reshape(NBLK),
                  slot2.reshape(NBLK), nxe2.reshape(NBLK),
                  hasnx2.reshape(NBLK), gx, fc1_w, fc1_b, fc2_w, fc2_b)
    out = _combine(y, pos0, pos1, w0r, w1r)

    return out.reshape(b, s, h), laux2.reshape(()), counts2.reshape(NE)
